# baseline (speedup 1.0000x reference)
import functools

import jax
import jax.numpy as jnp
from jax import lax
from jax.experimental import pallas as pl
from jax.experimental.pallas import tpu as pltpu
from jax.experimental.pallas import tpu_sc as plsc

EMB = 100000
DIM = 128
D2 = 2 * DIM
BATCH = 16384

NC = 2
NS = 16
NW = NC * NS
CHUNK = 64


def _sc_gather(pos_u, pos_w, W_word, W_ctx):
    n = pos_u.shape[0]
    bpw = n // NW
    nchunk = bpw // CHUNK
    mesh = plsc.VectorSubcoreMesh(
        core_axis_name="c", subcore_axis_name="s", num_cores=NC, num_subcores=NS
    )

    @functools.partial(
        pl.kernel,
        mesh=mesh,
        out_type=tuple(
            jax.ShapeDtypeStruct((n, DIM), jnp.float32) for _ in range(4)
        ),
        scratch_types=[
            pltpu.VMEM((bpw,), jnp.int32),
            pltpu.VMEM((bpw,), jnp.int32),
            pltpu.VMEM((CHUNK, D2), jnp.float32),
            pltpu.VMEM((CHUNK, D2), jnp.float32),
            pltpu.VMEM((CHUNK, D2), jnp.float32),
            pltpu.VMEM((CHUNK, D2), jnp.float32),
            pltpu.SemaphoreType.DMA,
            pltpu.SemaphoreType.DMA,
            pltpu.SemaphoreType.DMA,
        ],
    )
    def k(pu_hbm, pw_hbm, wu_hbm, wc_hbm, zu_hbm, du_hbm, zw_hbm, dw_hbm,
          idxu_v, idxw_v, ru0, ru1, rw0, rw1, gsemu, gsemw, osem):
        wid = lax.axis_index("s") * NC + lax.axis_index("c")
        base = wid * bpw
        rub = (ru0, ru1)
        rwb = (rw0, rw1)

        pltpu.sync_copy(pu_hbm.at[pl.ds(base, bpw)], idxu_v)
        pltpu.sync_copy(pw_hbm.at[pl.ds(base, bpw)], idxw_v)

        def start(i):
            b = i % 2
            isl = pl.ds(i * CHUNK, CHUNK)
            return (
                pltpu.async_copy(wu_hbm.at[idxu_v.at[isl]], rub[b], gsemu),
                pltpu.async_copy(wc_hbm.at[idxw_v.at[isl]], rwb[b], gsemw),
            )

        inflight = [start(0), start(1)]
        outcopies = []
        for i in range(nchunk):
            b = i % 2
            cu, cw = inflight[i]
            cu.wait()
            cw.wait()
            sl = pl.ds(base + i * CHUNK, CHUNK)
            occ = (
                pltpu.async_copy(rub[b].at[:, pl.ds(0, DIM)], zu_hbm.at[sl], osem),
                pltpu.async_copy(rub[b].at[:, pl.ds(DIM, DIM)], du_hbm.at[sl], osem),
                pltpu.async_copy(rwb[b].at[:, pl.ds(0, DIM)], zw_hbm.at[sl], osem),
                pltpu.async_copy(rwb[b].at[:, pl.ds(DIM, DIM)], dw_hbm.at[sl], osem),
            )
            outcopies.append(occ)
            if i + 2 < nchunk:
                for oc in occ:
                    oc.wait()
                inflight.append(start(i + 2))
        for occ in outcopies[-2:]:
            for oc in occ:
                oc.wait()

    return k(pos_u, pos_w, W_word, W_ctx)


def _lane_sum(x):
    return jnp.sum(x, axis=-1)


def _tc_body(zu_ref, du_ref, zw_ref, dw_ref, tv_ref, pv_ref, iv_ref):
    zu = zu_ref[...]
    zw = zw_ref[...]
    eu = jnp.exp(du_ref[...])
    ew = jnp.exp(dw_ref[...])
    tv_ref[...] = _lane_sum(jnp.log(jnp.log(2.0 + eu)))
    pv_ref[...] = _lane_sum(jnp.log(jnp.log(2.0 + ew)))
    xu = jnp.exp(zu)
    xw = jnp.exp(zw)
    r = jnp.minimum(xu * (1.0 + eu), xw * (1.0 + ew)) / jnp.maximum(xu, xw)
    iv_ref[...] = _lane_sum(jnp.log(jnp.log1p(r) + 1e-23))


_TC_ROWS = 2048


def _tc_compute(zu, du, zw, dw):
    n = zu.shape[0]
    grid = (n // _TC_ROWS,)
    return pl.pallas_call(
        _tc_body,
        grid=grid,
        in_specs=[pl.BlockSpec((_TC_ROWS, DIM), lambda i: (i, 0))] * 4,
        out_specs=[pl.BlockSpec((_TC_ROWS,), lambda i: (i,))] * 3,
        out_shape=[jax.ShapeDtypeStruct((n,), jnp.float32)] * 3,
    )(zu, du, zw, dw)


_NSLICE = 4


def kernel(pos_u, pos_w, W_word, W_ctx):
    ns = BATCH // _NSLICE
    parts = []
    for s in range(_NSLICE):
        pu = lax.slice_in_dim(pos_u, s * ns, (s + 1) * ns)
        pw = lax.slice_in_dim(pos_w, s * ns, (s + 1) * ns)
        parts.append(_tc_compute(*_sc_gather(pu, pw, W_word, W_ctx)))
    return tuple(
        jnp.concatenate([p[j] for p in parts]) for j in range(3)
    )


# speedup vs baseline: 1.0854x; 1.0854x over previous
import functools

import jax
import jax.numpy as jnp
from jax import lax
from jax.experimental import pallas as pl
from jax.experimental.pallas import tpu as pltpu
from jax.experimental.pallas import tpu_sc as plsc

EMB = 100000
DIM = 128
D2 = 2 * DIM
BATCH = 16384

NC = 2
NS = 16
NW = NC * NS
CHUNK = 64


def _sc_gather(pos_u, pos_w, W_word, W_ctx, s, n):
    bpw = n // NW
    nchunk = bpw // CHUNK
    mesh = plsc.VectorSubcoreMesh(
        core_axis_name="c", subcore_axis_name="s", num_cores=NC, num_subcores=NS
    )

    @functools.partial(
        pl.kernel,
        mesh=mesh,
        out_type=tuple(
            jax.ShapeDtypeStruct((n, DIM), jnp.float32) for _ in range(4)
        ),
        scratch_types=[
            pltpu.VMEM((bpw,), jnp.int32),
            pltpu.VMEM((bpw,), jnp.int32),
            pltpu.VMEM((CHUNK, D2), jnp.float32),
            pltpu.VMEM((CHUNK, D2), jnp.float32),
            pltpu.VMEM((CHUNK, D2), jnp.float32),
            pltpu.VMEM((CHUNK, D2), jnp.float32),
            pltpu.SemaphoreType.DMA,
            pltpu.SemaphoreType.DMA,
            pltpu.SemaphoreType.DMA,
        ],
    )
    def k(pu_hbm, pw_hbm, wu_hbm, wc_hbm, zu_hbm, du_hbm, zw_hbm, dw_hbm,
          idxu_v, idxw_v, ru0, ru1, rw0, rw1, gsemu, gsemw, osem):
        wid = lax.axis_index("s") * NC + lax.axis_index("c")
        base = wid * bpw
        rub = (ru0, ru1)
        rwb = (rw0, rw1)

        pltpu.sync_copy(pu_hbm.at[pl.ds(s * n + base, bpw)], idxu_v)
        pltpu.sync_copy(pw_hbm.at[pl.ds(s * n + base, bpw)], idxw_v)

        def start(i):
            b = i % 2
            isl = pl.ds(i * CHUNK, CHUNK)
            return (
                pltpu.async_copy(wu_hbm.at[idxu_v.at[isl]], rub[b], gsemu),
                pltpu.async_copy(wc_hbm.at[idxw_v.at[isl]], rwb[b], gsemw),
            )

        inflight = [start(0), start(1)]
        outcopies = []
        for i in range(nchunk):
            b = i % 2
            cu, cw = inflight[i]
            cu.wait()
            cw.wait()
            sl = pl.ds(base + i * CHUNK, CHUNK)
            occ = (
                pltpu.async_copy(rub[b].at[:, pl.ds(0, DIM)], zu_hbm.at[sl], osem),
                pltpu.async_copy(rub[b].at[:, pl.ds(DIM, DIM)], du_hbm.at[sl], osem),
                pltpu.async_copy(rwb[b].at[:, pl.ds(0, DIM)], zw_hbm.at[sl], osem),
                pltpu.async_copy(rwb[b].at[:, pl.ds(DIM, DIM)], dw_hbm.at[sl], osem),
            )
            outcopies.append(occ)
            if i + 2 < nchunk:
                for oc in occ:
                    oc.wait()
                inflight.append(start(i + 2))
        for occ in outcopies[-2:]:
            for oc in occ:
                oc.wait()

    return k(pos_u, pos_w, W_word, W_ctx)


def _lane_sum(x):
    return jnp.sum(x, axis=-1)


def _tc_body(zu_ref, du_ref, zw_ref, dw_ref, *rest):
    tv_ref, pv_ref, iv_ref = rest[-3:]
    zu = zu_ref[...]
    zw = zw_ref[...]
    eu = jnp.exp(du_ref[...])
    ew = jnp.exp(dw_ref[...])
    tv_ref[...] = _lane_sum(jnp.log(jnp.log(2.0 + eu)))
    pv_ref[...] = _lane_sum(jnp.log(jnp.log(2.0 + ew)))
    xu = jnp.exp(zu)
    xw = jnp.exp(zw)
    r = jnp.minimum(xu * (1.0 + eu), xw * (1.0 + ew)) / jnp.maximum(xu, xw)
    iv_ref[...] = _lane_sum(jnp.log(jnp.log1p(r) + 1e-23))


_TC_ROWS = 2048


def _tc_compute(zu, du, zw, dw, s, n, prev):
    grid = (n // _TC_ROWS,)
    blk = n // _TC_ROWS

    def out_map(i, _s=s, _blk=blk):
        return (i + _s * _blk,)

    in_specs = [pl.BlockSpec((_TC_ROWS, DIM), lambda i: (i, 0))] * 4
    args = [zu, du, zw, dw]
    aliases = {}
    if prev is not None:
        in_specs += [pl.BlockSpec((_TC_ROWS,), out_map)] * 3
        args += list(prev)
        aliases = {4: 0, 5: 1, 6: 2}
    return pl.pallas_call(
        _tc_body,
        grid=grid,
        in_specs=in_specs,
        out_specs=[pl.BlockSpec((_TC_ROWS,), out_map)] * 3,
        out_shape=[jax.ShapeDtypeStruct((BATCH,), jnp.float32)] * 3,
        input_output_aliases=aliases,
    )(*args)


_NSLICE = 2


def kernel(pos_u, pos_w, W_word, W_ctx):
    ns = BATCH // _NSLICE
    outs = None
    for s in range(_NSLICE):
        parts = _sc_gather(pos_u, pos_w, W_word, W_ctx, s, ns)
        outs = _tc_compute(*parts, s, ns, outs)
    return outs


# speedup vs baseline: 1.2519x; 1.1534x over previous
import functools

import jax
import jax.numpy as jnp
from jax import lax
from jax.experimental import pallas as pl
from jax.experimental.pallas import tpu as pltpu
from jax.experimental.pallas import tpu_sc as plsc

EMB = 100000
DIM = 128
D2 = 2 * DIM
BATCH = 16384

NC = 2
NS = 16
NW = NC * NS
CHUNK = 64


def _sc_gather(pos_u, pos_w, W_word, W_ctx, s, n):
    bpw = n // NW
    nchunk = bpw // CHUNK
    mesh = plsc.VectorSubcoreMesh(
        core_axis_name="c", subcore_axis_name="s", num_cores=NC, num_subcores=NS
    )

    @functools.partial(
        pl.kernel,
        mesh=mesh,
        out_type=tuple(
            jax.ShapeDtypeStruct((n, DIM), jnp.float32) for _ in range(4)
        ),
        scratch_types=[
            pltpu.VMEM((bpw,), jnp.int32),
            pltpu.VMEM((bpw,), jnp.int32),
            pltpu.VMEM((CHUNK, D2), jnp.float32),
            pltpu.VMEM((CHUNK, D2), jnp.float32),
            pltpu.VMEM((CHUNK, D2), jnp.float32),
            pltpu.VMEM((CHUNK, D2), jnp.float32),
            pltpu.SemaphoreType.DMA,
            pltpu.SemaphoreType.DMA,
            pltpu.SemaphoreType.DMA,
        ],
    )
    def k(pu_hbm, pw_hbm, wu_hbm, wc_hbm, zu_hbm, du_hbm, zw_hbm, dw_hbm,
          idxu_v, idxw_v, ru0, ru1, rw0, rw1, gsemu, gsemw, osem):
        wid = lax.axis_index("s") * NC + lax.axis_index("c")
        base = wid * bpw
        rub = (ru0, ru1)
        rwb = (rw0, rw1)

        pltpu.sync_copy(pu_hbm.at[pl.ds(s * n + base, bpw)], idxu_v)
        pltpu.sync_copy(pw_hbm.at[pl.ds(s * n + base, bpw)], idxw_v)

        def start(i):
            b = i % 2
            isl = pl.ds(i * CHUNK, CHUNK)
            return (
                pltpu.async_copy(wu_hbm.at[idxu_v.at[isl]], rub[b], gsemu),
                pltpu.async_copy(wc_hbm.at[idxw_v.at[isl]], rwb[b], gsemw),
            )

        inflight = [start(0), start(1)]
        outcopies = []
        for i in range(nchunk):
            b = i % 2
            cu, cw = inflight[i]
            cu.wait()
            cw.wait()
            sl = pl.ds(base + i * CHUNK, CHUNK)
            occ = (
                pltpu.async_copy(rub[b].at[:, pl.ds(0, DIM)], zu_hbm.at[sl], osem),
                pltpu.async_copy(rub[b].at[:, pl.ds(DIM, DIM)], du_hbm.at[sl], osem),
                pltpu.async_copy(rwb[b].at[:, pl.ds(0, DIM)], zw_hbm.at[sl], osem),
                pltpu.async_copy(rwb[b].at[:, pl.ds(DIM, DIM)], dw_hbm.at[sl], osem),
            )
            outcopies.append(occ)
            if i + 2 < nchunk:
                for oc in occ:
                    oc.wait()
                inflight.append(start(i + 2))
        for occ in outcopies[-2:]:
            for oc in occ:
                oc.wait()

    return k(pos_u, pos_w, W_word, W_ctx)


def _lane_sum(x):
    return jnp.sum(jnp.transpose(x), axis=0)


def _tc_body(zu_ref, du_ref, zw_ref, dw_ref, *rest):
    tv_ref, pv_ref, iv_ref = rest[-3:]
    zu = zu_ref[...]
    zw = zw_ref[...]
    eu = jnp.exp(du_ref[...])
    ew = jnp.exp(dw_ref[...])
    tv_ref[...] = _lane_sum(jnp.log(jnp.log(2.0 + eu)))
    pv_ref[...] = _lane_sum(jnp.log(jnp.log(2.0 + ew)))
    xu = jnp.exp(zu)
    xw = jnp.exp(zw)
    r = jnp.minimum(xu * (1.0 + eu), xw * (1.0 + ew)) / jnp.maximum(xu, xw)
    iv_ref[...] = _lane_sum(jnp.log(jnp.log1p(r) + 1e-23))


_TC_ROWS = 2048


def _tc_compute(zu, du, zw, dw, s, n, prev):
    grid = (n // _TC_ROWS,)
    blk = n // _TC_ROWS

    def out_map(i, _s=s, _blk=blk):
        return (i + _s * _blk,)

    in_specs = [pl.BlockSpec((_TC_ROWS, DIM), lambda i: (i, 0))] * 4
    args = [zu, du, zw, dw]
    aliases = {}
    if prev is not None:
        in_specs += [pl.BlockSpec((_TC_ROWS,), out_map)] * 3
        args += list(prev)
        aliases = {4: 0, 5: 1, 6: 2}
    return pl.pallas_call(
        _tc_body,
        grid=grid,
        in_specs=in_specs,
        out_specs=[pl.BlockSpec((_TC_ROWS,), out_map)] * 3,
        out_shape=[jax.ShapeDtypeStruct((BATCH,), jnp.float32)] * 3,
        input_output_aliases=aliases,
    )(*args)


_NSLICE = 2


def kernel(pos_u, pos_w, W_word, W_ctx):
    ns = BATCH // _NSLICE
    outs = None
    for s in range(_NSLICE):
        parts = _sc_gather(pos_u, pos_w, W_word, W_ctx, s, ns)
        outs = _tc_compute(*parts, s, ns, outs)
    return outs


# speedup vs baseline: 1.2568x; 1.0039x over previous
import functools

import jax
import jax.numpy as jnp
from jax import lax
from jax.experimental import pallas as pl
from jax.experimental.pallas import tpu as pltpu
from jax.experimental.pallas import tpu_sc as plsc

EMB = 100000
DIM = 128
D2 = 2 * DIM
BATCH = 16384

NC = 2
NS = 16
NW = NC * NS
CHUNK = 64


def _sc_gather(pos_u, pos_w, W_word, W_ctx, s, n):
    bpw = n // NW
    nchunk = bpw // CHUNK
    mesh = plsc.VectorSubcoreMesh(
        core_axis_name="c", subcore_axis_name="s", num_cores=NC, num_subcores=NS
    )

    @functools.partial(
        pl.kernel,
        mesh=mesh,
        out_type=tuple(
            jax.ShapeDtypeStruct((n, DIM), jnp.float32) for _ in range(4)
        ),
        scratch_types=[
            pltpu.VMEM((bpw,), jnp.int32),
            pltpu.VMEM((bpw,), jnp.int32),
            pltpu.VMEM((CHUNK, D2), jnp.float32),
            pltpu.VMEM((CHUNK, D2), jnp.float32),
            pltpu.VMEM((CHUNK, D2), jnp.float32),
            pltpu.VMEM((CHUNK, D2), jnp.float32),
            pltpu.VMEM((CHUNK, D2), jnp.float32),
            pltpu.VMEM((CHUNK, D2), jnp.float32),
            pltpu.SemaphoreType.DMA,
            pltpu.SemaphoreType.DMA,
            pltpu.SemaphoreType.DMA,
        ],
    )
    def k(pu_hbm, pw_hbm, wu_hbm, wc_hbm, zu_hbm, du_hbm, zw_hbm, dw_hbm,
          idxu_v, idxw_v, ru0, ru1, ru2, rw0, rw1, rw2, gsemu, gsemw, osem):
        wid = lax.axis_index("s") * NC + lax.axis_index("c")
        base = wid * bpw
        rub = (ru0, ru1, ru2)
        rwb = (rw0, rw1, rw2)

        pltpu.sync_copy(pu_hbm.at[pl.ds(s * n + base, bpw)], idxu_v)
        pltpu.sync_copy(pw_hbm.at[pl.ds(s * n + base, bpw)], idxw_v)

        nbuf = 3

        def start(i):
            b = i % nbuf
            isl = pl.ds(i * CHUNK, CHUNK)
            return (
                pltpu.async_copy(wu_hbm.at[idxu_v.at[isl]], rub[b], gsemu),
                pltpu.async_copy(wc_hbm.at[idxw_v.at[isl]], rwb[b], gsemw),
            )

        inflight = [start(i) for i in range(min(nbuf, nchunk))]
        outcopies = []
        for i in range(nchunk):
            b = i % nbuf
            cu, cw = inflight[i]
            cu.wait()
            cw.wait()
            sl = pl.ds(base + i * CHUNK, CHUNK)
            occ = (
                pltpu.async_copy(rub[b].at[:, pl.ds(0, DIM)], zu_hbm.at[sl], osem),
                pltpu.async_copy(rub[b].at[:, pl.ds(DIM, DIM)], du_hbm.at[sl], osem),
                pltpu.async_copy(rwb[b].at[:, pl.ds(0, DIM)], zw_hbm.at[sl], osem),
                pltpu.async_copy(rwb[b].at[:, pl.ds(DIM, DIM)], dw_hbm.at[sl], osem),
            )
            outcopies.append(occ)
            if i + nbuf < nchunk:
                for oc in occ:
                    oc.wait()
                inflight.append(start(i + nbuf))
        for occ in outcopies[-nbuf:]:
            for oc in occ:
                oc.wait()

    return k(pos_u, pos_w, W_word, W_ctx)


def _lane_sum(x):
    return jnp.sum(jnp.transpose(x), axis=0)


def _tc_body(zu_ref, du_ref, zw_ref, dw_ref, *rest):
    tv_ref, pv_ref, iv_ref = rest[-3:]
    zu = zu_ref[...]
    zw = zw_ref[...]
    eu = jnp.exp(du_ref[...])
    ew = jnp.exp(dw_ref[...])
    tv_ref[...] = _lane_sum(jnp.log(jnp.log(2.0 + eu)))
    pv_ref[...] = _lane_sum(jnp.log(jnp.log(2.0 + ew)))
    xu = jnp.exp(zu)
    xw = jnp.exp(zw)
    r = jnp.minimum(xu * (1.0 + eu), xw * (1.0 + ew)) / jnp.maximum(xu, xw)
    iv_ref[...] = _lane_sum(jnp.log(jnp.log1p(r) + 1e-23))


_TC_ROWS = 2048


def _tc_compute(zu, du, zw, dw, s, n, prev):
    grid = (n // _TC_ROWS,)
    blk = n // _TC_ROWS

    def out_map(i, _s=s, _blk=blk):
        return (i + _s * _blk,)

    in_specs = [pl.BlockSpec((_TC_ROWS, DIM), lambda i: (i, 0))] * 4
    args = [zu, du, zw, dw]
    aliases = {}
    if prev is not None:
        in_specs += [pl.BlockSpec((_TC_ROWS,), out_map)] * 3
        args += list(prev)
        aliases = {4: 0, 5: 1, 6: 2}
    return pl.pallas_call(
        _tc_body,
        grid=grid,
        in_specs=in_specs,
        out_specs=[pl.BlockSpec((_TC_ROWS,), out_map)] * 3,
        out_shape=[jax.ShapeDtypeStruct((BATCH,), jnp.float32)] * 3,
        input_output_aliases=aliases,
    )(*args)


_NSLICE = 2


def kernel(pos_u, pos_w, W_word, W_ctx):
    ns = BATCH // _NSLICE
    outs = None
    for s in range(_NSLICE):
        parts = _sc_gather(pos_u, pos_w, W_word, W_ctx, s, ns)
        outs = _tc_compute(*parts, s, ns, outs)
    return outs
